# baseline (speedup 1.0000x reference)
import functools

import jax
import jax.numpy as jnp
from jax import lax
from jax.experimental import pallas as pl
from jax.experimental.pallas import tpu as pltpu
from jax.experimental.pallas import tpu_sc as plsc

N = 10000
E = 640000
D_IN = 768
H = 128

NT = 32
RPT = 313
NPAD = NT * RPT
TPAD = NPAD + 320
ACCR = 320
KE = 256
KH = KE // 2
SENT = 768
CH = 8000
FL = 8192
SS = 16384
LCAP = E + FL + 1024

_f32 = jnp.float32
_i32 = jnp.int32


def _proj_tail(xl, asv, adv, asr_ref, adr_ref, m_ref, first):
    dn = (((1,), (1,)), ((), ()))
    asr = lax.dot_general(asv, xl, dn, preferred_element_type=_f32)
    adr = lax.dot_general(adv, xl, dn, preferred_element_type=_f32)
    asr_ref[...] = asr[None]
    adr_ref[...] = adr[None]
    es = asr + adr
    es = jnp.where(es > 0.0, es, 0.2 * es)
    mb = jnp.max(es)

    @pl.when(first)
    def _():
        m_ref[...] = jnp.full((1, 1), -jnp.inf, _f32)

    m_ref[...] = jnp.maximum(m_ref[...], mb)


def _tc1_body(x_ref, w_ref, asv_ref, adv_ref, xl_ref, asr_ref, adr_ref, m_ref):
    xl = jnp.dot(x_ref[...], w_ref[...], preferred_element_type=_f32)
    xl_ref[...] = xl
    _proj_tail(xl, asv_ref[...], adv_ref[...], asr_ref, adr_ref, m_ref,
               pl.program_id(0) == 0)


def _tc1(x, W, a_src, a_dst):
    BM = 1000
    G = N // BM
    return pl.pallas_call(
        _tc1_body,
        grid=(G,),
        in_specs=[
            pl.BlockSpec((BM, x.shape[1]), lambda i: (i, 0)),
            pl.BlockSpec((x.shape[1], H), lambda i: (0, 0)),
            pl.BlockSpec((1, H), lambda i: (0, 0)),
            pl.BlockSpec((1, H), lambda i: (0, 0)),
        ],
        out_specs=[
            pl.BlockSpec((BM, H), lambda i: (i, 0)),
            pl.BlockSpec((1, 1, BM), lambda i: (i, 0, 0)),
            pl.BlockSpec((1, 1, BM), lambda i: (i, 0, 0)),
            pl.BlockSpec((1, 1), lambda i: (0, 0)),
        ],
        out_shape=[
            jax.ShapeDtypeStruct((N, H), _f32),
            jax.ShapeDtypeStruct((G, 1, BM), _f32),
            jax.ShapeDtypeStruct((G, 1, BM), _f32),
            jax.ShapeDtypeStruct((1, 1), _f32),
        ],
    )(x, W, a_src.reshape(1, H), a_dst.reshape(1, H))


def _norm_h(acc, den, xl, asv, adv, m, b):
    asc = jnp.sum(xl * asv, axis=1, keepdims=True)
    adc = jnp.sum(xl * adv, axis=1, keepdims=True)
    es = asc + adc
    es = jnp.where(es > 0.0, es, 0.2 * es)
    exs = jnp.exp(es - m)
    return (acc + exs * xl) / (den + exs + 1e-16) + b


def _tc2_body(acc_ref, den_ref, xl_ref, as1_ref, ad1_ref, m1_ref, b1_ref,
              w_ref, asv_ref, adv_ref, xl2_ref, asr_ref, adr_ref, m2_ref):
    h = _norm_h(acc_ref[...], den_ref[...], xl_ref[...], as1_ref[...],
                ad1_ref[...], m1_ref[0, 0], b1_ref[...])
    h = jnp.where(h > 0.0, h, jnp.exp(jnp.minimum(h, 0.0)) - 1.0)
    xl2 = jnp.dot(h, w_ref[...], preferred_element_type=_f32)
    xl2_ref[...] = xl2
    _proj_tail(xl2, asv_ref[...], adv_ref[...], asr_ref, adr_ref, m2_ref,
               pl.program_id(0) == 0)


def _tc2(acc, den_b, xl1, a_src1, a_dst1, m1, b1, W2, a_src2, a_dst2):
    BM = 1000
    G = N // BM
    vec = pl.BlockSpec((1, H), lambda i: (0, 0))
    blk = pl.BlockSpec((BM, H), lambda i: (i, 0))
    return pl.pallas_call(
        _tc2_body,
        grid=(G,),
        in_specs=[blk, blk, blk, vec, vec,
                  pl.BlockSpec((1, 1), lambda i: (0, 0)), vec,
                  pl.BlockSpec((H, H), lambda i: (0, 0)), vec, vec],
        out_specs=[
            pl.BlockSpec((BM, H), lambda i: (i, 0)),
            pl.BlockSpec((1, 1, BM), lambda i: (i, 0, 0)),
            pl.BlockSpec((1, 1, BM), lambda i: (i, 0, 0)),
            pl.BlockSpec((1, 1), lambda i: (0, 0)),
        ],
        out_shape=[
            jax.ShapeDtypeStruct((N, H), _f32),
            jax.ShapeDtypeStruct((G, 1, BM), _f32),
            jax.ShapeDtypeStruct((G, 1, BM), _f32),
            jax.ShapeDtypeStruct((1, 1), _f32),
        ],
    )(acc, den_b, xl1, a_src1.reshape(1, H), a_dst1.reshape(1, H), m1,
      b1.reshape(1, H), W2, a_src2.reshape(1, H), a_dst2.reshape(1, H))


def _tc3_body(acc_ref, den_ref, xl_ref, as2_ref, ad2_ref, m2_ref, b2_ref,
              wp_ref, bp_ref, out_ref):
    h = _norm_h(acc_ref[...], den_ref[...], xl_ref[...], as2_ref[...],
                ad2_ref[...], m2_ref[0, 0], b2_ref[...])
    o = jnp.dot(h, wp_ref[...], preferred_element_type=_f32) + bp_ref[...]
    out_ref[...] = jnp.maximum(o, 0.0)


def _tc3(acc, den_b, xl2, a_src2, a_dst2, m2, b2, Wp, bp):
    BM = 1000
    G = N // BM
    vec = pl.BlockSpec((1, H), lambda i: (0, 0))
    blk = pl.BlockSpec((BM, H), lambda i: (i, 0))
    return pl.pallas_call(
        _tc3_body,
        grid=(G,),
        in_specs=[blk, blk, blk, vec, vec,
                  pl.BlockSpec((1, 1), lambda i: (0, 0)), vec,
                  pl.BlockSpec((H, H), lambda i: (0, 0)), vec],
        out_specs=pl.BlockSpec((BM, H), lambda i: (i, 0)),
        out_shape=jax.ShapeDtypeStruct((N, H), _f32),
    )(acc, den_b, xl2, a_src2.reshape(1, H), a_dst2.reshape(1, H), m2,
      b2.reshape(1, H), Wp, bp.reshape(1, H))


def _wid():
    return lax.axis_index("s") * 2 + lax.axis_index("c")


def _filter_body(src_ref, dst_ref, slist_ref, dlist_ref, counts_ref,
                 sbufA, dbufA, sbufB, dbufB, stg_s, stg_d, cbuf, semA, semB):
    wid = _wid()
    lo = wid * RPT
    iot = lax.iota(_i32, 16)
    zero16 = jnp.zeros((16,), _i32)
    sent = jnp.full((16,), RPT, _i32)
    NCHF = E // CH
    NPF = NCHF // 2

    def _issue(ci, sb, db, sem):
        off = pl.multiple_of(ci * CH, 8)
        pltpu.async_copy(src_ref.at[pl.ds(off, CH)], sb, sem)
        pltpu.async_copy(dst_ref.at[pl.ds(off, CH)], db, sem)

    def _drain(sb, db, sem):
        pltpu.make_async_copy(src_ref.at[pl.ds(0, CH)], sb, sem).wait()
        pltpu.make_async_copy(dst_ref.at[pl.ds(0, CH)], db, sem).wait()

    def _flush(cl, wt):
        fo = pl.multiple_of(wid * LCAP + wt, 8)
        pltpu.sync_copy(stg_s.at[pl.ds(0, FL)], slist_ref.at[pl.ds(fo, FL)])
        pltpu.sync_copy(stg_d.at[pl.ds(0, FL)], dlist_ref.at[pl.ds(fo, FL)])

        @plsc.parallel_loop(0, FL // 16, 1, unroll=4)
        def _mv(t):
            idx = jnp.full((16,), FL + t * 16, _i32) + iot
            dstx = jnp.full((16,), t * 16, _i32) + iot
            plsc.store_scatter(stg_s, [dstx], plsc.load_gather(stg_s, [idx]))
            plsc.store_scatter(stg_d, [dstx], plsc.load_gather(stg_d, [idx]))
        return cl - FL, wt + FL

    def _noflush(cl, wt):
        return cl, wt

    def _append(sb, db, cl, wt):
        clv = jnp.full((16,), cl, _i32)

        def _t(t, clv):
            t80 = jnp.full((16,), t * 80, _i32)
            for u in range(5):
                idx = t80 + (u * 16 + iot)
                sv = plsc.load_gather(sb, [idx])
                dv = plsc.load_gather(db, [idx])
                dloc = dv - lo
                msk = (dloc >= 0) & (dloc < RPT)
                pos = clv + plsc.cumsum(msk.astype(_i32)) - 1
                plsc.store_scatter(stg_s, [pos], sv, mask=msk)
                plsc.store_scatter(stg_d, [pos], dloc, mask=msk)
                clv = clv + plsc.all_reduce_population_count(msk)
            return clv

        clv = plsc.parallel_loop(0, CH // 80, 1, unroll=2, carry=clv)(_t)
        cl = jnp.max(clv)
        return lax.cond(cl >= FL, _flush, _noflush, cl, wt)

    _issue(0, sbufA, dbufA, semA)
    _issue(1, sbufB, dbufB, semB)

    def _pair(p, carry):
        cl, wt = carry
        _drain(sbufA, dbufA, semA)
        cl, wt = _append(sbufA, dbufA, cl, wt)

        @pl.when(2 * p + 2 < NCHF)
        def _():
            _issue(2 * p + 2, sbufA, dbufA, semA)

        _drain(sbufB, dbufB, semB)
        cl, wt = _append(sbufB, dbufB, cl, wt)

        @pl.when(2 * p + 3 < NCHF)
        def _():
            _issue(2 * p + 3, sbufB, dbufB, semB)

        return cl, wt

    cl, wt = lax.fori_loop(0, NPF, _pair, (jnp.int32(0), jnp.int32(0)))
    tcount = wt + cl
    for t in range(SENT // 16):
        pos = cl + t * 16 + iot
        plsc.store_scatter(stg_s, [pos], zero16)
        plsc.store_scatter(stg_d, [pos], sent)
    cl = cl + SENT
    cl, wt = lax.cond(cl >= FL, _flush, _noflush, cl, wt)
    fo = pl.multiple_of(wid * LCAP + wt, 8)
    pltpu.sync_copy(stg_s.at[pl.ds(0, FL)], slist_ref.at[pl.ds(fo, FL)])
    pltpu.sync_copy(stg_d.at[pl.ds(0, FL)], dlist_ref.at[pl.ds(fo, FL)])
    cbuf[...] = jnp.where(iot == 0, jnp.full((16,), tcount, _i32), 0)
    pltpu.sync_copy(cbuf, counts_ref.at[pl.ds(pl.multiple_of(wid * 16, 8), 16)])


@functools.partial(
    pl.kernel,
    out_type=[
        jax.ShapeDtypeStruct((NT * LCAP,), _i32),
        jax.ShapeDtypeStruct((NT * LCAP,), _i32),
        jax.ShapeDtypeStruct((NT * 16,), _i32),
    ],
    mesh=plsc.VectorSubcoreMesh(core_axis_name="c", subcore_axis_name="s"),
    compiler_params=pltpu.CompilerParams(needs_layout_passes=False),
    scratch_types=[
        pltpu.VMEM((CH,), _i32),
        pltpu.VMEM((CH,), _i32),
        pltpu.VMEM((CH,), _i32),
        pltpu.VMEM((CH,), _i32),
        pltpu.VMEM((SS,), _i32),
        pltpu.VMEM((SS,), _i32),
        pltpu.VMEM((16,), _i32),
        pltpu.SemaphoreType.DMA,
        pltpu.SemaphoreType.DMA,
    ],
)
def _filter(src_ref, dst_ref, slist_ref, dlist_ref, counts_ref, *scr):
    _filter_body(src_ref, dst_ref, slist_ref, dlist_ref, counts_ref, *scr)


def _edge_body(slist_ref, dlist_ref, counts_ref, asq_ref, adq_ref, xl_ref,
               mv_ref, acc_ref, dens_ref,
               as_t, ad_t, mbuf, cbuf,
               slb0, dlb0, exb0, rows0, slb1, dlb1, exb1, rows1,
               accv, denv, semL0, semL1, semR0, semR1):
    wid = _wid()
    lo = wid * RPT
    iot = lax.iota(_i32, 16)
    lane0 = iot == 0
    zero16f = jnp.zeros((16,), _f32)
    offs = [jj * 16 + iot for jj in range(H // 16)]

    pltpu.sync_copy(asq_ref, as_t)
    pltpu.sync_copy(adq_ref, ad_t)
    pltpu.sync_copy(mv_ref, mbuf)
    pltpu.sync_copy(counts_ref.at[pl.ds(pl.multiple_of(wid * 16, 8), 16)], cbuf)

    @plsc.parallel_loop(0, ACCR * H // 16, 1, unroll=8)
    def _zero(i):
        plsc.store_scatter(accv, [jnp.full((16,), i * 16, _i32) + iot], zero16f)
    for t in range(ACCR // 16):
        denv[pl.ds(t * 16, 16)] = zero16f

    cnt = jnp.max(plsc.load_gather(cbuf, [jnp.zeros((16,), _i32)]))
    nch = (cnt + (KE - 1)) // KE
    npair = (nch + 1) // 2
    mval = mbuf[...]

    def _issue_lists(ci, sb, db, sem):
        base = pl.multiple_of(wid * LCAP + ci * KE, 8)
        pltpu.async_copy(slist_ref.at[pl.ds(base, KE)], sb, sem)
        pltpu.async_copy(dlist_ref.at[pl.ds(base, KE)], db, sem)

    def _drain_lists(sb, db, sem):
        pltpu.make_async_copy(slist_ref.at[pl.ds(0, KE)], sb, sem).wait()
        pltpu.make_async_copy(dlist_ref.at[pl.ds(0, KE)], db, sem).wait()

    def _issue_rows(sb, rows, sem):
        pltpu.async_copy(xl_ref.at[sb.at[pl.ds(0, KH)]],
                         rows.at[pl.ds(0, KH)], sem)
        pltpu.async_copy(xl_ref.at[sb.at[pl.ds(KH, KH)]],
                         rows.at[pl.ds(KH, KH)], sem)

    def _drain_rows(sb, rows, sem):
        pltpu.make_async_copy(xl_ref.at[sb.at[pl.ds(0, KH)]],
                              rows.at[pl.ds(0, KH)], sem).wait()
        pltpu.make_async_copy(xl_ref.at[sb.at[pl.ds(KH, KH)]],
                              rows.at[pl.ds(KH, KH)], sem).wait()

    ucs = [iot == u for u in range(16)]
    ninf = jnp.full((16,), -jnp.inf, _f32)
    zv = jnp.zeros((16,), _i32)

    def _compute(sb, db, eb, rows):
        del eb

        @plsc.parallel_loop(0, KE // 16, 1, unroll=2)
        def _grp(g):
            g16 = jnp.full((16,), g * 16, _i32)
            sv = plsc.load_gather(sb, [g16 + iot])
            dv = plsc.load_gather(db, [g16 + iot])
            asg = plsc.load_gather(as_t, [sv])
            adg = plsc.load_gather(ad_t, [dv + lo])
            e = asg + adg
            e = jnp.where(e > 0.0, e, 0.2 * e)
            exv = jnp.exp(e - mval)
            rbv = dv * H
            for u in range(16):
                plsc.addupdate_scatter(denv, [dv], exv, mask=ucs[u])
                exs = jnp.max(jnp.where(ucs[u], exv, ninf))
                rb = pl.multiple_of(jnp.max(jnp.where(ucs[u], rbv, zv)), 8)
                js = g16 + u
                for jj in range(H // 16):
                    rv = plsc.load_gather(rows, [js, offs[jj]])
                    plsc.addupdate(accv.at[pl.ds(rb + jj * 16, 16)], rv * exs)

    _issue_lists(0, slb0, dlb0, semL0)
    _drain_lists(slb0, dlb0, semL0)
    _issue_rows(slb0, rows0, semR0)
    _issue_lists(1, slb1, dlb1, semL1)

    def _pair(p, _):
        cA = 2 * p
        _drain_rows(slb0, rows0, semR0)
        _drain_lists(slb1, dlb1, semL1)
        _issue_rows(slb1, rows1, semR1)
        _compute(slb0, dlb0, exb0, rows0)
        _issue_lists(cA + 2, slb0, dlb0, semL0)
        _drain_rows(slb1, rows1, semR1)
        _drain_lists(slb0, dlb0, semL0)
        _issue_rows(slb0, rows0, semR0)
        _compute(slb1, dlb1, exb1, rows1)
        _issue_lists(cA + 3, slb1, dlb1, semL1)
        return 0

    lax.fori_loop(0, npair, _pair, 0)
    _drain_rows(slb0, rows0, semR0)
    _drain_lists(slb1, dlb1, semL1)

    pltpu.sync_copy(accv.at[pl.ds(0, RPT * H)],
                    acc_ref.at[pl.ds(pl.multiple_of(lo * H, 8), RPT * H)])
    pltpu.sync_copy(denv, dens_ref.at[pl.ds(pl.multiple_of(wid * ACCR, 8), ACCR)])


@functools.partial(
    pl.kernel,
    out_type=[
        jax.ShapeDtypeStruct((NPAD * H,), _f32),
        jax.ShapeDtypeStruct((NT * ACCR,), _f32),
    ],
    mesh=plsc.VectorSubcoreMesh(core_axis_name="c", subcore_axis_name="s"),
    compiler_params=pltpu.CompilerParams(needs_layout_passes=False),
    scratch_types=[
        pltpu.VMEM((TPAD,), _f32),
        pltpu.VMEM((TPAD,), _f32),
        pltpu.VMEM((16,), _f32),
        pltpu.VMEM((16,), _i32),
        pltpu.VMEM((KE,), _i32),
        pltpu.VMEM((KE,), _i32),
        pltpu.VMEM((KE,), _f32),
        pltpu.VMEM((KE, H), _f32),
        pltpu.VMEM((KE,), _i32),
        pltpu.VMEM((KE,), _i32),
        pltpu.VMEM((KE,), _f32),
        pltpu.VMEM((KE, H), _f32),
        pltpu.VMEM((ACCR * H,), _f32),
        pltpu.VMEM((ACCR,), _f32),
        pltpu.SemaphoreType.DMA,
        pltpu.SemaphoreType.DMA,
        pltpu.SemaphoreType.DMA,
        pltpu.SemaphoreType.DMA,
    ],
)
def _edge_pass(slist_ref, dlist_ref, counts_ref, asq_ref, adq_ref, xl_ref,
               mv_ref, acc_ref, dens_ref, *scr):
    _edge_body(slist_ref, dlist_ref, counts_ref, asq_ref, adq_ref, xl_ref,
               mv_ref, acc_ref, dens_ref, *scr)


def _pad_table(v):
    return jnp.pad(v.reshape(-1), (0, TPAD - N))


def _sc_layer(slist, dlist, counts, asr, adr, xl, m):
    asq = _pad_table(asr)
    adq = _pad_table(adr)
    mv = jnp.broadcast_to(m.reshape(()), (16,))
    accf, densf = _edge_pass(slist, dlist, counts, asq, adq, xl, mv)
    acc = accf.reshape(NPAD, H)[:N]
    den = densf.reshape(NT, ACCR)[:, :RPT].reshape(NPAD)[:N]
    den_b = jnp.broadcast_to(den[:, None], (N, H))
    return acc, den_b


def kernel(x, edge_index, W1, a_src1, a_dst1, b1, W2, a_src2, a_dst2, b2,
           Wp, bp):
    src = edge_index[0]
    dst = edge_index[1]
    slist, dlist, counts = _filter(src, dst)

    xl1, asr1, adr1, m1 = _tc1(x, W1, a_src1, a_dst1)
    acc1, den1b = _sc_layer(slist, dlist, counts, asr1, adr1, xl1, m1)

    xl2, asr2, adr2, m2 = _tc2(acc1, den1b, xl1, a_src1, a_dst1, m1, b1,
                               W2, a_src2, a_dst2)
    acc2, den2b = _sc_layer(slist, dlist, counts, asr2, adr2, xl2, m2)

    return _tc3(acc2, den2b, xl2, a_src2, a_dst2, m2, b2, Wp, bp)


# speedup vs baseline: 1.0811x; 1.0811x over previous
import functools

import jax
import jax.numpy as jnp
from jax import lax
from jax.experimental import pallas as pl
from jax.experimental.pallas import tpu as pltpu
from jax.experimental.pallas import tpu_sc as plsc

N = 10000
E = 640000
D_IN = 768
H = 128

NT = 32
RPT = 313
NPAD = NT * RPT
TPAD = NPAD + 320
ACCR = 320
KE = 256
KH = KE // 2
SENT = 768
CH = 8000
FL = 8192
SS = 16384
LCAP = E + FL + 1024

_f32 = jnp.float32
_i32 = jnp.int32


def _proj_tail(xl, asv, adv, asr_ref, adr_ref, m_ref, first):
    dn = (((1,), (1,)), ((), ()))
    asr = lax.dot_general(asv, xl, dn, preferred_element_type=_f32)
    adr = lax.dot_general(adv, xl, dn, preferred_element_type=_f32)
    asr_ref[...] = asr[None]
    adr_ref[...] = adr[None]
    es = asr + adr
    es = jnp.where(es > 0.0, es, 0.2 * es)
    mb = jnp.max(es)

    @pl.when(first)
    def _():
        m_ref[...] = jnp.full((1, 1), -jnp.inf, _f32)

    m_ref[...] = jnp.maximum(m_ref[...], mb)


def _tc1_body(x_ref, w_ref, asv_ref, adv_ref, xl_ref, asr_ref, adr_ref, m_ref):
    xl = jnp.dot(x_ref[...], w_ref[...], preferred_element_type=_f32)
    xl_ref[...] = xl
    _proj_tail(xl, asv_ref[...], adv_ref[...], asr_ref, adr_ref, m_ref,
               pl.program_id(0) == 0)


def _tc1(x, W, a_src, a_dst):
    BM = 1000
    G = N // BM
    return pl.pallas_call(
        _tc1_body,
        grid=(G,),
        in_specs=[
            pl.BlockSpec((BM, x.shape[1]), lambda i: (i, 0)),
            pl.BlockSpec((x.shape[1], H), lambda i: (0, 0)),
            pl.BlockSpec((1, H), lambda i: (0, 0)),
            pl.BlockSpec((1, H), lambda i: (0, 0)),
        ],
        out_specs=[
            pl.BlockSpec((BM, H), lambda i: (i, 0)),
            pl.BlockSpec((1, 1, BM), lambda i: (i, 0, 0)),
            pl.BlockSpec((1, 1, BM), lambda i: (i, 0, 0)),
            pl.BlockSpec((1, 1), lambda i: (0, 0)),
        ],
        out_shape=[
            jax.ShapeDtypeStruct((N, H), _f32),
            jax.ShapeDtypeStruct((G, 1, BM), _f32),
            jax.ShapeDtypeStruct((G, 1, BM), _f32),
            jax.ShapeDtypeStruct((1, 1), _f32),
        ],
    )(x, W, a_src.reshape(1, H), a_dst.reshape(1, H))


def _norm_h(acc, den, xl, asv, adv, m, b):
    asc = jnp.sum(xl * asv, axis=1, keepdims=True)
    adc = jnp.sum(xl * adv, axis=1, keepdims=True)
    es = asc + adc
    es = jnp.where(es > 0.0, es, 0.2 * es)
    exs = jnp.exp(es - m)
    return (acc + exs * xl) / (den + exs + 1e-16) + b


def _tc2_body(acc_ref, den_ref, xl_ref, as1_ref, ad1_ref, m1_ref, b1_ref,
              w_ref, asv_ref, adv_ref, xl2_ref, asr_ref, adr_ref, m2_ref):
    h = _norm_h(acc_ref[...], den_ref[...], xl_ref[...], as1_ref[...],
                ad1_ref[...], m1_ref[0, 0], b1_ref[...])
    h = jnp.where(h > 0.0, h, jnp.exp(jnp.minimum(h, 0.0)) - 1.0)
    xl2 = jnp.dot(h, w_ref[...], preferred_element_type=_f32)
    xl2_ref[...] = xl2
    _proj_tail(xl2, asv_ref[...], adv_ref[...], asr_ref, adr_ref, m2_ref,
               pl.program_id(0) == 0)


def _tc2(acc, den_b, xl1, a_src1, a_dst1, m1, b1, W2, a_src2, a_dst2):
    BM = 1000
    G = N // BM
    vec = pl.BlockSpec((1, H), lambda i: (0, 0))
    blk = pl.BlockSpec((BM, H), lambda i: (i, 0))
    return pl.pallas_call(
        _tc2_body,
        grid=(G,),
        in_specs=[blk, blk, blk, vec, vec,
                  pl.BlockSpec((1, 1), lambda i: (0, 0)), vec,
                  pl.BlockSpec((H, H), lambda i: (0, 0)), vec, vec],
        out_specs=[
            pl.BlockSpec((BM, H), lambda i: (i, 0)),
            pl.BlockSpec((1, 1, BM), lambda i: (i, 0, 0)),
            pl.BlockSpec((1, 1, BM), lambda i: (i, 0, 0)),
            pl.BlockSpec((1, 1), lambda i: (0, 0)),
        ],
        out_shape=[
            jax.ShapeDtypeStruct((N, H), _f32),
            jax.ShapeDtypeStruct((G, 1, BM), _f32),
            jax.ShapeDtypeStruct((G, 1, BM), _f32),
            jax.ShapeDtypeStruct((1, 1), _f32),
        ],
    )(acc, den_b, xl1, a_src1.reshape(1, H), a_dst1.reshape(1, H), m1,
      b1.reshape(1, H), W2, a_src2.reshape(1, H), a_dst2.reshape(1, H))


def _tc3_body(acc_ref, den_ref, xl_ref, as2_ref, ad2_ref, m2_ref, b2_ref,
              wp_ref, bp_ref, out_ref):
    h = _norm_h(acc_ref[...], den_ref[...], xl_ref[...], as2_ref[...],
                ad2_ref[...], m2_ref[0, 0], b2_ref[...])
    o = jnp.dot(h, wp_ref[...], preferred_element_type=_f32) + bp_ref[...]
    out_ref[...] = jnp.maximum(o, 0.0)


def _tc3(acc, den_b, xl2, a_src2, a_dst2, m2, b2, Wp, bp):
    BM = 1000
    G = N // BM
    vec = pl.BlockSpec((1, H), lambda i: (0, 0))
    blk = pl.BlockSpec((BM, H), lambda i: (i, 0))
    return pl.pallas_call(
        _tc3_body,
        grid=(G,),
        in_specs=[blk, blk, blk, vec, vec,
                  pl.BlockSpec((1, 1), lambda i: (0, 0)), vec,
                  pl.BlockSpec((H, H), lambda i: (0, 0)), vec],
        out_specs=pl.BlockSpec((BM, H), lambda i: (i, 0)),
        out_shape=jax.ShapeDtypeStruct((N, H), _f32),
    )(acc, den_b, xl2, a_src2.reshape(1, H), a_dst2.reshape(1, H), m2,
      b2.reshape(1, H), Wp, bp.reshape(1, H))


def _wid():
    return lax.axis_index("s") * 2 + lax.axis_index("c")


def _filter_body(src_ref, dst_ref, slist_ref, dlist_ref, counts_ref,
                 sbufA, dbufA, sbufB, dbufB, stg_s, stg_d, cbuf, semA, semB):
    wid = _wid()
    lo = wid * RPT
    iot = lax.iota(_i32, 16)
    zero16 = jnp.zeros((16,), _i32)
    sent = jnp.full((16,), RPT, _i32)
    NCHF = E // CH
    NPF = NCHF // 2

    def _issue(ci, sb, db, sem):
        off = pl.multiple_of(ci * CH, 8)
        pltpu.async_copy(src_ref.at[pl.ds(off, CH)], sb, sem)
        pltpu.async_copy(dst_ref.at[pl.ds(off, CH)], db, sem)

    def _drain(sb, db, sem):
        pltpu.make_async_copy(src_ref.at[pl.ds(0, CH)], sb, sem).wait()
        pltpu.make_async_copy(dst_ref.at[pl.ds(0, CH)], db, sem).wait()

    def _flush(cl, wt):
        fo = pl.multiple_of(wid * LCAP + wt, 8)
        pltpu.sync_copy(stg_s.at[pl.ds(0, FL)], slist_ref.at[pl.ds(fo, FL)])
        pltpu.sync_copy(stg_d.at[pl.ds(0, FL)], dlist_ref.at[pl.ds(fo, FL)])

        @plsc.parallel_loop(0, FL // 16, 1, unroll=4)
        def _mv(t):
            idx = jnp.full((16,), FL + t * 16, _i32) + iot
            dstx = jnp.full((16,), t * 16, _i32) + iot
            plsc.store_scatter(stg_s, [dstx], plsc.load_gather(stg_s, [idx]))
            plsc.store_scatter(stg_d, [dstx], plsc.load_gather(stg_d, [idx]))
        return cl - FL, wt + FL

    def _noflush(cl, wt):
        return cl, wt

    def _append(sb, db, cl, wt):
        clv = jnp.full((16,), cl, _i32)

        def _t(t, clv):
            t80 = jnp.full((16,), t * 80, _i32)
            for u in range(5):
                idx = t80 + (u * 16 + iot)
                sv = plsc.load_gather(sb, [idx])
                dv = plsc.load_gather(db, [idx])
                dloc = dv - lo
                msk = (dloc >= 0) & (dloc < RPT)
                pos = clv + plsc.cumsum(msk.astype(_i32)) - 1
                plsc.store_scatter(stg_s, [pos], sv, mask=msk)
                plsc.store_scatter(stg_d, [pos], dloc, mask=msk)
                clv = clv + plsc.all_reduce_population_count(msk)
            return clv

        clv = plsc.parallel_loop(0, CH // 80, 1, unroll=2, carry=clv)(_t)
        cl = jnp.max(clv)
        return lax.cond(cl >= FL, _flush, _noflush, cl, wt)

    _issue(0, sbufA, dbufA, semA)
    _issue(1, sbufB, dbufB, semB)

    def _pair(p, carry):
        cl, wt = carry
        _drain(sbufA, dbufA, semA)
        cl, wt = _append(sbufA, dbufA, cl, wt)

        @pl.when(2 * p + 2 < NCHF)
        def _():
            _issue(2 * p + 2, sbufA, dbufA, semA)

        _drain(sbufB, dbufB, semB)
        cl, wt = _append(sbufB, dbufB, cl, wt)

        @pl.when(2 * p + 3 < NCHF)
        def _():
            _issue(2 * p + 3, sbufB, dbufB, semB)

        return cl, wt

    cl, wt = lax.fori_loop(0, NPF, _pair, (jnp.int32(0), jnp.int32(0)))
    tcount = wt + cl
    for t in range(SENT // 16):
        pos = cl + t * 16 + iot
        plsc.store_scatter(stg_s, [pos], zero16)
        plsc.store_scatter(stg_d, [pos], sent)
    cl = cl + SENT
    cl, wt = lax.cond(cl >= FL, _flush, _noflush, cl, wt)
    fo = pl.multiple_of(wid * LCAP + wt, 8)
    pltpu.sync_copy(stg_s.at[pl.ds(0, FL)], slist_ref.at[pl.ds(fo, FL)])
    pltpu.sync_copy(stg_d.at[pl.ds(0, FL)], dlist_ref.at[pl.ds(fo, FL)])
    cbuf[...] = jnp.where(iot == 0, jnp.full((16,), tcount, _i32), 0)
    pltpu.sync_copy(cbuf, counts_ref.at[pl.ds(pl.multiple_of(wid * 16, 8), 16)])


@functools.partial(
    pl.kernel,
    out_type=[
        jax.ShapeDtypeStruct((NT * LCAP,), _i32),
        jax.ShapeDtypeStruct((NT * LCAP,), _i32),
        jax.ShapeDtypeStruct((NT * 16,), _i32),
    ],
    mesh=plsc.VectorSubcoreMesh(core_axis_name="c", subcore_axis_name="s"),
    compiler_params=pltpu.CompilerParams(needs_layout_passes=False),
    scratch_types=[
        pltpu.VMEM((CH,), _i32),
        pltpu.VMEM((CH,), _i32),
        pltpu.VMEM((CH,), _i32),
        pltpu.VMEM((CH,), _i32),
        pltpu.VMEM((SS,), _i32),
        pltpu.VMEM((SS,), _i32),
        pltpu.VMEM((16,), _i32),
        pltpu.SemaphoreType.DMA,
        pltpu.SemaphoreType.DMA,
    ],
)
def _filter(src_ref, dst_ref, slist_ref, dlist_ref, counts_ref, *scr):
    _filter_body(src_ref, dst_ref, slist_ref, dlist_ref, counts_ref, *scr)


def _edge_body(slist_ref, dlist_ref, counts_ref, asq_ref, adq_ref, xl_ref,
               mv_ref, acc_ref, dens_ref,
               as_t, ad_t, mbuf, cbuf,
               slb0, dlb0, exb0, rows0, slb1, dlb1, exb1, rows1,
               accv, denv, semL0, semL1, semR0, semR1):
    wid = _wid()
    lo = wid * RPT
    iot = lax.iota(_i32, 16)
    lane0 = iot == 0
    zero16f = jnp.zeros((16,), _f32)
    offs = [jj * 16 + iot for jj in range(H // 16)]

    pltpu.sync_copy(asq_ref, as_t)
    pltpu.sync_copy(adq_ref, ad_t)
    pltpu.sync_copy(mv_ref, mbuf)
    pltpu.sync_copy(counts_ref.at[pl.ds(pl.multiple_of(wid * 16, 8), 16)], cbuf)

    @plsc.parallel_loop(0, ACCR * H // 16, 1, unroll=8)
    def _zero(i):
        plsc.store_scatter(accv, [jnp.full((16,), i * 16, _i32) + iot], zero16f)
    for t in range(ACCR // 16):
        denv[pl.ds(t * 16, 16)] = zero16f

    cnt = jnp.max(plsc.load_gather(cbuf, [jnp.zeros((16,), _i32)]))
    nch = (cnt + (KE - 1)) // KE
    npair = (nch + 1) // 2
    mval = mbuf[...]

    def _issue_lists(ci, sb, db, sem):
        base = pl.multiple_of(wid * LCAP + ci * KE, 8)
        pltpu.async_copy(slist_ref.at[pl.ds(base, KE)], sb, sem)
        pltpu.async_copy(dlist_ref.at[pl.ds(base, KE)], db, sem)

    def _drain_lists(sb, db, sem):
        pltpu.make_async_copy(slist_ref.at[pl.ds(0, KE)], sb, sem).wait()
        pltpu.make_async_copy(dlist_ref.at[pl.ds(0, KE)], db, sem).wait()

    def _issue_rows(sb, rows, sem):
        pltpu.async_copy(xl_ref.at[sb.at[pl.ds(0, KH)]],
                         rows.at[pl.ds(0, KH)], sem)
        pltpu.async_copy(xl_ref.at[sb.at[pl.ds(KH, KH)]],
                         rows.at[pl.ds(KH, KH)], sem)

    def _drain_rows(sb, rows, sem):
        pltpu.make_async_copy(xl_ref.at[sb.at[pl.ds(0, KH)]],
                              rows.at[pl.ds(0, KH)], sem).wait()
        pltpu.make_async_copy(xl_ref.at[sb.at[pl.ds(KH, KH)]],
                              rows.at[pl.ds(KH, KH)], sem).wait()

    ucs = [iot == u for u in range(16)]

    def _compute(sb, db, eb, rows):
        @plsc.parallel_loop(0, KE // 16, 1, unroll=2)
        def _grp(g):
            g16 = jnp.full((16,), g * 16, _i32)
            sv = plsc.load_gather(sb, [g16 + iot])
            dv = plsc.load_gather(db, [g16 + iot])
            asg = plsc.load_gather(as_t, [sv])
            adg = plsc.load_gather(ad_t, [dv + lo])
            e = asg + adg
            e = jnp.where(e > 0.0, e, 0.2 * e)
            exv = jnp.exp(e - mval)
            plsc.store_scatter(eb, [g16 + iot], exv)
            for u in range(16):
                js = g16 + u
                dls = plsc.load_gather(db, [js])
                exs = plsc.load_gather(eb, [js])
                plsc.addupdate_scatter(denv, [dv], exv, mask=ucs[u])
                rbase = dls * H
                for jj in range(H // 16):
                    rv = plsc.load_gather(rows, [js, offs[jj]])
                    plsc.addupdate_scatter(accv, [rbase + offs[jj]], rv * exs)

    _issue_lists(0, slb0, dlb0, semL0)
    _drain_lists(slb0, dlb0, semL0)
    _issue_rows(slb0, rows0, semR0)
    _issue_lists(1, slb1, dlb1, semL1)

    def _pair(p, _):
        cA = 2 * p
        _drain_rows(slb0, rows0, semR0)
        _drain_lists(slb1, dlb1, semL1)
        _issue_rows(slb1, rows1, semR1)
        _compute(slb0, dlb0, exb0, rows0)
        _issue_lists(cA + 2, slb0, dlb0, semL0)
        _drain_rows(slb1, rows1, semR1)
        _drain_lists(slb0, dlb0, semL0)
        _issue_rows(slb0, rows0, semR0)
        _compute(slb1, dlb1, exb1, rows1)
        _issue_lists(cA + 3, slb1, dlb1, semL1)
        return 0

    lax.fori_loop(0, npair, _pair, 0)
    _drain_rows(slb0, rows0, semR0)
    _drain_lists(slb1, dlb1, semL1)

    pltpu.sync_copy(accv.at[pl.ds(0, RPT * H)],
                    acc_ref.at[pl.ds(pl.multiple_of(lo * H, 8), RPT * H)])
    pltpu.sync_copy(denv, dens_ref.at[pl.ds(pl.multiple_of(wid * ACCR, 8), ACCR)])


@functools.partial(
    pl.kernel,
    out_type=[
        jax.ShapeDtypeStruct((NPAD * H,), _f32),
        jax.ShapeDtypeStruct((NT * ACCR,), _f32),
    ],
    mesh=plsc.VectorSubcoreMesh(core_axis_name="c", subcore_axis_name="s"),
    compiler_params=pltpu.CompilerParams(needs_layout_passes=False),
    scratch_types=[
        pltpu.VMEM((TPAD,), _f32),
        pltpu.VMEM((TPAD,), _f32),
        pltpu.VMEM((16,), _f32),
        pltpu.VMEM((16,), _i32),
        pltpu.VMEM((KE,), _i32),
        pltpu.VMEM((KE,), _i32),
        pltpu.VMEM((KE,), _f32),
        pltpu.VMEM((KE, H), _f32),
        pltpu.VMEM((KE,), _i32),
        pltpu.VMEM((KE,), _i32),
        pltpu.VMEM((KE,), _f32),
        pltpu.VMEM((KE, H), _f32),
        pltpu.VMEM((ACCR * H,), _f32),
        pltpu.VMEM((ACCR,), _f32),
        pltpu.SemaphoreType.DMA,
        pltpu.SemaphoreType.DMA,
        pltpu.SemaphoreType.DMA,
        pltpu.SemaphoreType.DMA,
    ],
)
def _edge_pass(slist_ref, dlist_ref, counts_ref, asq_ref, adq_ref, xl_ref,
               mv_ref, acc_ref, dens_ref, *scr):
    _edge_body(slist_ref, dlist_ref, counts_ref, asq_ref, adq_ref, xl_ref,
               mv_ref, acc_ref, dens_ref, *scr)


def _pad_table(v):
    return jnp.pad(v.reshape(-1), (0, TPAD - N))


def _sc_layer(slist, dlist, counts, asr, adr, xl, m):
    asq = _pad_table(asr)
    adq = _pad_table(adr)
    mv = jnp.broadcast_to(m.reshape(()), (16,))
    accf, densf = _edge_pass(slist, dlist, counts, asq, adq, xl, mv)
    acc = accf.reshape(NPAD, H)[:N]
    den = densf.reshape(NT, ACCR)[:, :RPT].reshape(NPAD)[:N]
    den_b = jnp.broadcast_to(den[:, None], (N, H))
    return acc, den_b


def kernel(x, edge_index, W1, a_src1, a_dst1, b1, W2, a_src2, a_dst2, b2,
           Wp, bp):
    src = edge_index[0]
    dst = edge_index[1]
    slist, dlist, counts = _filter(src, dst)

    xl1, asr1, adr1, m1 = _tc1(x, W1, a_src1, a_dst1)
    acc1, den1b = _sc_layer(slist, dlist, counts, asr1, adr1, xl1, m1)

    xl2, asr2, adr2, m2 = _tc2(acc1, den1b, xl1, a_src1, a_dst1, m1, b1,
                               W2, a_src2, a_dst2)
    acc2, den2b = _sc_layer(slist, dlist, counts, asr2, adr2, xl2, m2)

    return _tc3(acc2, den2b, xl2, a_src2, a_dst2, m2, b2, Wp, bp)


# speedup vs baseline: 1.3565x; 1.2548x over previous
import functools

import jax
import jax.numpy as jnp
from jax import lax
from jax.experimental import pallas as pl
from jax.experimental.pallas import tpu as pltpu
from jax.experimental.pallas import tpu_sc as plsc

N = 10000
E = 640000
D_IN = 768
H = 128

NT = 32
RPT = 313
NPAD = NT * RPT
TPAD = NPAD + 320
ACCR = 320
KE = 256
KH = KE // 2
SENT = 768
CH = 8000
FL = 8192
SS = 16384
LCAP = E + FL + 1024

_f32 = jnp.float32
_i32 = jnp.int32


def _proj_tail(xl, asv, adv, asr_ref, adr_ref, m_ref, first):
    dn = (((1,), (1,)), ((), ()))
    asr = lax.dot_general(asv, xl, dn, preferred_element_type=_f32)
    adr = lax.dot_general(adv, xl, dn, preferred_element_type=_f32)
    asr_ref[...] = asr[None]
    adr_ref[...] = adr[None]
    es = asr + adr
    es = jnp.where(es > 0.0, es, 0.2 * es)
    mb = jnp.max(es)

    @pl.when(first)
    def _():
        m_ref[...] = jnp.full((1, 1), -jnp.inf, _f32)

    m_ref[...] = jnp.maximum(m_ref[...], mb)


def _tc1_body(x_ref, w_ref, asv_ref, adv_ref, xl_ref, asr_ref, adr_ref, m_ref):
    xl = jnp.dot(x_ref[...], w_ref[...], preferred_element_type=_f32)
    xl_ref[...] = xl
    _proj_tail(xl, asv_ref[...], adv_ref[...], asr_ref, adr_ref, m_ref,
               pl.program_id(0) == 0)


def _tc1(x, W, a_src, a_dst):
    BM = 1000
    G = N // BM
    return pl.pallas_call(
        _tc1_body,
        grid=(G,),
        in_specs=[
            pl.BlockSpec((BM, x.shape[1]), lambda i: (i, 0)),
            pl.BlockSpec((x.shape[1], H), lambda i: (0, 0)),
            pl.BlockSpec((1, H), lambda i: (0, 0)),
            pl.BlockSpec((1, H), lambda i: (0, 0)),
        ],
        out_specs=[
            pl.BlockSpec((BM, H), lambda i: (i, 0)),
            pl.BlockSpec((1, 1, BM), lambda i: (i, 0, 0)),
            pl.BlockSpec((1, 1, BM), lambda i: (i, 0, 0)),
            pl.BlockSpec((1, 1), lambda i: (0, 0)),
        ],
        out_shape=[
            jax.ShapeDtypeStruct((N, H), _f32),
            jax.ShapeDtypeStruct((G, 1, BM), _f32),
            jax.ShapeDtypeStruct((G, 1, BM), _f32),
            jax.ShapeDtypeStruct((1, 1), _f32),
        ],
    )(x, W, a_src.reshape(1, H), a_dst.reshape(1, H))


def _norm_h(acc, den, xl, asv, adv, m, b):
    asc = jnp.sum(xl * asv, axis=1, keepdims=True)
    adc = jnp.sum(xl * adv, axis=1, keepdims=True)
    es = asc + adc
    es = jnp.where(es > 0.0, es, 0.2 * es)
    exs = jnp.exp(es - m)
    return (acc + exs * xl) / (den + exs + 1e-16) + b


def _tc2_body(acc_ref, den_ref, xl_ref, as1_ref, ad1_ref, m1_ref, b1_ref,
              w_ref, asv_ref, adv_ref, xl2_ref, asr_ref, adr_ref, m2_ref):
    h = _norm_h(acc_ref[...], den_ref[...], xl_ref[...], as1_ref[...],
                ad1_ref[...], m1_ref[0, 0], b1_ref[...])
    h = jnp.where(h > 0.0, h, jnp.exp(jnp.minimum(h, 0.0)) - 1.0)
    xl2 = jnp.dot(h, w_ref[...], preferred_element_type=_f32)
    xl2_ref[...] = xl2
    _proj_tail(xl2, asv_ref[...], adv_ref[...], asr_ref, adr_ref, m2_ref,
               pl.program_id(0) == 0)


def _tc2(acc, den_b, xl1, a_src1, a_dst1, m1, b1, W2, a_src2, a_dst2):
    BM = 1000
    G = N // BM
    vec = pl.BlockSpec((1, H), lambda i: (0, 0))
    blk = pl.BlockSpec((BM, H), lambda i: (i, 0))
    return pl.pallas_call(
        _tc2_body,
        grid=(G,),
        in_specs=[blk, blk, blk, vec, vec,
                  pl.BlockSpec((1, 1), lambda i: (0, 0)), vec,
                  pl.BlockSpec((H, H), lambda i: (0, 0)), vec, vec],
        out_specs=[
            pl.BlockSpec((BM, H), lambda i: (i, 0)),
            pl.BlockSpec((1, 1, BM), lambda i: (i, 0, 0)),
            pl.BlockSpec((1, 1, BM), lambda i: (i, 0, 0)),
            pl.BlockSpec((1, 1), lambda i: (0, 0)),
        ],
        out_shape=[
            jax.ShapeDtypeStruct((N, H), _f32),
            jax.ShapeDtypeStruct((G, 1, BM), _f32),
            jax.ShapeDtypeStruct((G, 1, BM), _f32),
            jax.ShapeDtypeStruct((1, 1), _f32),
        ],
    )(acc, den_b, xl1, a_src1.reshape(1, H), a_dst1.reshape(1, H), m1,
      b1.reshape(1, H), W2, a_src2.reshape(1, H), a_dst2.reshape(1, H))


def _tc3_body(acc_ref, den_ref, xl_ref, as2_ref, ad2_ref, m2_ref, b2_ref,
              wp_ref, bp_ref, out_ref):
    h = _norm_h(acc_ref[...], den_ref[...], xl_ref[...], as2_ref[...],
                ad2_ref[...], m2_ref[0, 0], b2_ref[...])
    o = jnp.dot(h, wp_ref[...], preferred_element_type=_f32) + bp_ref[...]
    out_ref[...] = jnp.maximum(o, 0.0)


def _tc3(acc, den_b, xl2, a_src2, a_dst2, m2, b2, Wp, bp):
    BM = 1000
    G = N // BM
    vec = pl.BlockSpec((1, H), lambda i: (0, 0))
    blk = pl.BlockSpec((BM, H), lambda i: (i, 0))
    return pl.pallas_call(
        _tc3_body,
        grid=(G,),
        in_specs=[blk, blk, blk, vec, vec,
                  pl.BlockSpec((1, 1), lambda i: (0, 0)), vec,
                  pl.BlockSpec((H, H), lambda i: (0, 0)), vec],
        out_specs=pl.BlockSpec((BM, H), lambda i: (i, 0)),
        out_shape=jax.ShapeDtypeStruct((N, H), _f32),
    )(acc, den_b, xl2, a_src2.reshape(1, H), a_dst2.reshape(1, H), m2,
      b2.reshape(1, H), Wp, bp.reshape(1, H))


def _wid():
    return lax.axis_index("s") * 2 + lax.axis_index("c")


def _filter_body(src_ref, dst_ref, slist_ref, dlist_ref, counts_ref,
                 sbufA, dbufA, sbufB, dbufB, stg_s, stg_d, cbuf, semA, semB):
    wid = _wid()
    lo = wid * RPT
    iot = lax.iota(_i32, 16)
    zero16 = jnp.zeros((16,), _i32)
    sent = jnp.full((16,), RPT, _i32)
    NCHF = E // CH
    NPF = NCHF // 2

    def _issue(ci, sb, db, sem):
        off = pl.multiple_of(ci * CH, 8)
        pltpu.async_copy(src_ref.at[pl.ds(off, CH)], sb, sem)
        pltpu.async_copy(dst_ref.at[pl.ds(off, CH)], db, sem)

    def _drain(sb, db, sem):
        pltpu.make_async_copy(src_ref.at[pl.ds(0, CH)], sb, sem).wait()
        pltpu.make_async_copy(dst_ref.at[pl.ds(0, CH)], db, sem).wait()

    def _flush(cl, wt):
        fo = pl.multiple_of(wid * LCAP + wt, 8)
        pltpu.sync_copy(stg_s.at[pl.ds(0, FL)], slist_ref.at[pl.ds(fo, FL)])
        pltpu.sync_copy(stg_d.at[pl.ds(0, FL)], dlist_ref.at[pl.ds(fo, FL)])

        @plsc.parallel_loop(0, FL // 16, 1, unroll=4)
        def _mv(t):
            idx = jnp.full((16,), FL + t * 16, _i32) + iot
            dstx = jnp.full((16,), t * 16, _i32) + iot
            plsc.store_scatter(stg_s, [dstx], plsc.load_gather(stg_s, [idx]))
            plsc.store_scatter(stg_d, [dstx], plsc.load_gather(stg_d, [idx]))
        return cl - FL, wt + FL

    def _noflush(cl, wt):
        return cl, wt

    def _append(sb, db, cl, wt):
        clv = jnp.full((16,), cl, _i32)

        def _t(t, clv):
            t80 = jnp.full((16,), t * 80, _i32)
            for u in range(5):
                idx = t80 + (u * 16 + iot)
                sv = plsc.load_gather(sb, [idx])
                dv = plsc.load_gather(db, [idx])
                dloc = dv - lo
                msk = (dloc >= 0) & (dloc < RPT)
                pos = clv + plsc.cumsum(msk.astype(_i32)) - 1
                plsc.store_scatter(stg_s, [pos], sv, mask=msk)
                plsc.store_scatter(stg_d, [pos], dloc, mask=msk)
                clv = clv + plsc.all_reduce_population_count(msk)
            return clv

        clv = plsc.parallel_loop(0, CH // 80, 1, unroll=2, carry=clv)(_t)
        cl = jnp.max(clv)
        return lax.cond(cl >= FL, _flush, _noflush, cl, wt)

    _issue(0, sbufA, dbufA, semA)
    _issue(1, sbufB, dbufB, semB)

    def _pair(p, carry):
        cl, wt = carry
        _drain(sbufA, dbufA, semA)
        cl, wt = _append(sbufA, dbufA, cl, wt)

        @pl.when(2 * p + 2 < NCHF)
        def _():
            _issue(2 * p + 2, sbufA, dbufA, semA)

        _drain(sbufB, dbufB, semB)
        cl, wt = _append(sbufB, dbufB, cl, wt)

        @pl.when(2 * p + 3 < NCHF)
        def _():
            _issue(2 * p + 3, sbufB, dbufB, semB)

        return cl, wt

    cl, wt = lax.fori_loop(0, NPF, _pair, (jnp.int32(0), jnp.int32(0)))
    tcount = wt + cl
    for t in range(SENT // 16):
        pos = cl + t * 16 + iot
        plsc.store_scatter(stg_s, [pos], zero16)
        plsc.store_scatter(stg_d, [pos], sent)
    cl = cl + SENT
    cl, wt = lax.cond(cl >= FL, _flush, _noflush, cl, wt)
    fo = pl.multiple_of(wid * LCAP + wt, 8)
    pltpu.sync_copy(stg_s.at[pl.ds(0, FL)], slist_ref.at[pl.ds(fo, FL)])
    pltpu.sync_copy(stg_d.at[pl.ds(0, FL)], dlist_ref.at[pl.ds(fo, FL)])
    cbuf[...] = jnp.where(iot == 0, jnp.full((16,), tcount, _i32), 0)
    pltpu.sync_copy(cbuf, counts_ref.at[pl.ds(pl.multiple_of(wid * 16, 8), 16)])


@functools.partial(
    pl.kernel,
    out_type=[
        jax.ShapeDtypeStruct((NT * LCAP,), _i32),
        jax.ShapeDtypeStruct((NT * LCAP,), _i32),
        jax.ShapeDtypeStruct((NT * 16,), _i32),
    ],
    mesh=plsc.VectorSubcoreMesh(core_axis_name="c", subcore_axis_name="s"),
    compiler_params=pltpu.CompilerParams(needs_layout_passes=False),
    scratch_types=[
        pltpu.VMEM((CH,), _i32),
        pltpu.VMEM((CH,), _i32),
        pltpu.VMEM((CH,), _i32),
        pltpu.VMEM((CH,), _i32),
        pltpu.VMEM((SS,), _i32),
        pltpu.VMEM((SS,), _i32),
        pltpu.VMEM((16,), _i32),
        pltpu.SemaphoreType.DMA,
        pltpu.SemaphoreType.DMA,
    ],
)
def _filter(src_ref, dst_ref, slist_ref, dlist_ref, counts_ref, *scr):
    _filter_body(src_ref, dst_ref, slist_ref, dlist_ref, counts_ref, *scr)


def _edge_body(slist_ref, dlist_ref, counts_ref, asq_ref, adq_ref, xl_ref,
               mv_ref, acc_ref, dens_ref,
               as_t, ad_t, mbuf, cbuf,
               slb0, dlb0, exb0, rows0, slb1, dlb1, exb1, rows1,
               accv, denv, semL0, semL1, semR0, semR1):
    wid = _wid()
    lo = wid * RPT
    iot = lax.iota(_i32, 16)
    lane0 = iot == 0
    zero16f = jnp.zeros((16,), _f32)
    offs = [jj * 16 + iot for jj in range(H // 16)]

    pltpu.sync_copy(asq_ref, as_t)
    pltpu.sync_copy(adq_ref, ad_t)
    pltpu.sync_copy(mv_ref, mbuf)
    pltpu.sync_copy(counts_ref.at[pl.ds(pl.multiple_of(wid * 16, 8), 16)], cbuf)

    @plsc.parallel_loop(0, ACCR * H // 16, 1, unroll=8)
    def _zero(i):
        plsc.store_scatter(accv, [jnp.full((16,), i * 16, _i32) + iot], zero16f)
    for t in range(ACCR // 16):
        denv[pl.ds(t * 16, 16)] = zero16f

    cnt = jnp.max(plsc.load_gather(cbuf, [jnp.zeros((16,), _i32)]))
    nch = (cnt + (KE - 1)) // KE
    npair = (nch + 1) // 2
    mval = mbuf[...]

    def _issue_lists(ci, sb, db, sem):
        base = pl.multiple_of(wid * LCAP + ci * KE, 8)
        pltpu.async_copy(slist_ref.at[pl.ds(base, KE)], sb, sem)
        pltpu.async_copy(dlist_ref.at[pl.ds(base, KE)], db, sem)

    def _drain_lists(sb, db, sem):
        pltpu.make_async_copy(slist_ref.at[pl.ds(0, KE)], sb, sem).wait()
        pltpu.make_async_copy(dlist_ref.at[pl.ds(0, KE)], db, sem).wait()

    def _issue_rows(sb, rows, sem):
        pltpu.async_copy(xl_ref.at[sb.at[pl.ds(0, KH)]],
                         rows.at[pl.ds(0, KH)], sem)
        pltpu.async_copy(xl_ref.at[sb.at[pl.ds(KH, KH)]],
                         rows.at[pl.ds(KH, KH)], sem)

    def _drain_rows(sb, rows, sem):
        pltpu.make_async_copy(xl_ref.at[sb.at[pl.ds(0, KH)]],
                              rows.at[pl.ds(0, KH)], sem).wait()
        pltpu.make_async_copy(xl_ref.at[sb.at[pl.ds(KH, KH)]],
                              rows.at[pl.ds(KH, KH)], sem).wait()

    def _compute(sb, db, eb, rows):
        for t in range(KE // 16):
            sv = sb[pl.ds(t * 16, 16)]
            dv = db[pl.ds(t * 16, 16)]
            asg = plsc.load_gather(as_t, [sv])
            adg = plsc.load_gather(ad_t, [dv + lo])
            e = asg + adg
            e = jnp.where(e > 0.0, e, 0.2 * e)
            eb[pl.ds(t * 16, 16)] = jnp.exp(e - mval)

        @plsc.parallel_loop(0, KE, 1, unroll=8)
        def _edge1(j):
            js = jnp.full((16,), j, _i32)
            dls = plsc.load_gather(db, [js])
            exs = plsc.load_gather(eb, [js])
            plsc.addupdate_scatter(denv, [dls], exs, mask=lane0)
            rbase = dls * H
            for jj in range(H // 16):
                rv = plsc.load_gather(rows, [js, offs[jj]])
                plsc.addupdate_scatter(accv, [rbase + offs[jj]], rv * exs)

    _issue_lists(0, slb0, dlb0, semL0)
    _drain_lists(slb0, dlb0, semL0)
    _issue_rows(slb0, rows0, semR0)
    _issue_lists(1, slb1, dlb1, semL1)

    def _pair(p, _):
        cA = 2 * p
        _drain_rows(slb0, rows0, semR0)
        _drain_lists(slb1, dlb1, semL1)
        _issue_rows(slb1, rows1, semR1)
        _compute(slb0, dlb0, exb0, rows0)
        _issue_lists(cA + 2, slb0, dlb0, semL0)
        _drain_rows(slb1, rows1, semR1)
        _drain_lists(slb0, dlb0, semL0)
        _issue_rows(slb0, rows0, semR0)
        _compute(slb1, dlb1, exb1, rows1)
        _issue_lists(cA + 3, slb1, dlb1, semL1)
        return 0

    lax.fori_loop(0, npair, _pair, 0)
    _drain_rows(slb0, rows0, semR0)
    _drain_lists(slb1, dlb1, semL1)

    pltpu.sync_copy(accv.at[pl.ds(0, RPT * H)],
                    acc_ref.at[pl.ds(pl.multiple_of(lo * H, 8), RPT * H)])
    pltpu.sync_copy(denv, dens_ref.at[pl.ds(pl.multiple_of(wid * ACCR, 8), ACCR)])


@functools.partial(
    pl.kernel,
    out_type=[
        jax.ShapeDtypeStruct((NPAD * H,), _f32),
        jax.ShapeDtypeStruct((NT * ACCR,), _f32),
    ],
    mesh=plsc.VectorSubcoreMesh(core_axis_name="c", subcore_axis_name="s"),
    compiler_params=pltpu.CompilerParams(needs_layout_passes=False),
    scratch_types=[
        pltpu.VMEM((TPAD,), _f32),
        pltpu.VMEM((TPAD,), _f32),
        pltpu.VMEM((16,), _f32),
        pltpu.VMEM((16,), _i32),
        pltpu.VMEM((KE,), _i32),
        pltpu.VMEM((KE,), _i32),
        pltpu.VMEM((KE,), _f32),
        pltpu.VMEM((KE, H), _f32),
        pltpu.VMEM((KE,), _i32),
        pltpu.VMEM((KE,), _i32),
        pltpu.VMEM((KE,), _f32),
        pltpu.VMEM((KE, H), _f32),
        pltpu.VMEM((ACCR * H,), _f32),
        pltpu.VMEM((ACCR,), _f32),
        pltpu.SemaphoreType.DMA,
        pltpu.SemaphoreType.DMA,
        pltpu.SemaphoreType.DMA,
        pltpu.SemaphoreType.DMA,
    ],
)
def _edge_pass(slist_ref, dlist_ref, counts_ref, asq_ref, adq_ref, xl_ref,
               mv_ref, acc_ref, dens_ref, *scr):
    _edge_body(slist_ref, dlist_ref, counts_ref, asq_ref, adq_ref, xl_ref,
               mv_ref, acc_ref, dens_ref, *scr)


def _pad_table(v):
    return jnp.pad(v.reshape(-1), (0, TPAD - N))


def _sc_layer(slist, dlist, counts, asr, adr, xl, m):
    asq = _pad_table(asr)
    adq = _pad_table(adr)
    mv = jnp.broadcast_to(m.reshape(()), (16,))
    accf, densf = _edge_pass(slist, dlist, counts, asq, adq, xl, mv)
    acc = accf.reshape(NPAD, H)[:N]
    den = densf.reshape(NT, ACCR)[:, :RPT].reshape(NPAD)[:N]
    den_b = jnp.broadcast_to(den[:, None], (N, H))
    return acc, den_b


def kernel(x, edge_index, W1, a_src1, a_dst1, b1, W2, a_src2, a_dst2, b2,
           Wp, bp):
    src = edge_index[0]
    dst = edge_index[1]
    slist, dlist, counts = _filter(src, dst)

    xl1, asr1, adr1, m1 = _tc1(x, W1, a_src1, a_dst1)
    acc1, den1b = _sc_layer(slist, dlist, counts, asr1, adr1, xl1, m1)

    xl2, asr2, adr2, m2 = _tc2(acc1, den1b, xl1, a_src1, a_dst1, m1, b1,
                               W2, a_src2, a_dst2)
    acc2, den2b = _sc_layer(slist, dlist, counts, asr2, adr2, xl2, m2)

    return _tc3(acc2, den2b, xl2, a_src2, a_dst2, m2, b2, Wp, bp)


# speedup vs baseline: 1.3616x; 1.0037x over previous
import functools

import jax
import jax.numpy as jnp
from jax import lax
from jax.experimental import pallas as pl
from jax.experimental.pallas import tpu as pltpu
from jax.experimental.pallas import tpu_sc as plsc

N = 10000
E = 640000
D_IN = 768
H = 128

NT = 32
RPT = 313
NPAD = NT * RPT
TPAD = NPAD + 320
ACCR = 320
KE = 256
KH = KE // 2
SENT = 768
CH = 8000
FL = 8192
SS = 16384
LCAP = E + FL + 1024

_f32 = jnp.float32
_i32 = jnp.int32


def _proj_tail(xl, asv, adv, asr_ref, adr_ref, m_ref, first):
    dn = (((1,), (1,)), ((), ()))
    asr = lax.dot_general(asv, xl, dn, preferred_element_type=_f32)
    adr = lax.dot_general(adv, xl, dn, preferred_element_type=_f32)
    asr_ref[...] = asr[None]
    adr_ref[...] = adr[None]
    es = asr + adr
    es = jnp.where(es > 0.0, es, 0.2 * es)
    mb = jnp.max(es)

    @pl.when(first)
    def _():
        m_ref[...] = jnp.full((1, 1), -jnp.inf, _f32)

    m_ref[...] = jnp.maximum(m_ref[...], mb)


def _tc1_body(x_ref, w_ref, asv_ref, adv_ref, xl_ref, asr_ref, adr_ref, m_ref):
    xl = jnp.dot(x_ref[...], w_ref[...], preferred_element_type=_f32)
    xl_ref[...] = xl
    _proj_tail(xl, asv_ref[...], adv_ref[...], asr_ref, adr_ref, m_ref,
               pl.program_id(0) == 0)


def _tc1(x, W, a_src, a_dst):
    BM = 1000
    G = N // BM
    return pl.pallas_call(
        _tc1_body,
        grid=(G,),
        in_specs=[
            pl.BlockSpec((BM, x.shape[1]), lambda i: (i, 0)),
            pl.BlockSpec((x.shape[1], H), lambda i: (0, 0)),
            pl.BlockSpec((1, H), lambda i: (0, 0)),
            pl.BlockSpec((1, H), lambda i: (0, 0)),
        ],
        out_specs=[
            pl.BlockSpec((BM, H), lambda i: (i, 0)),
            pl.BlockSpec((1, 1, BM), lambda i: (i, 0, 0)),
            pl.BlockSpec((1, 1, BM), lambda i: (i, 0, 0)),
            pl.BlockSpec((1, 1), lambda i: (0, 0)),
        ],
        out_shape=[
            jax.ShapeDtypeStruct((N, H), _f32),
            jax.ShapeDtypeStruct((G, 1, BM), _f32),
            jax.ShapeDtypeStruct((G, 1, BM), _f32),
            jax.ShapeDtypeStruct((1, 1), _f32),
        ],
    )(x, W, a_src.reshape(1, H), a_dst.reshape(1, H))


def _norm_h(acc, den, xl, asv, adv, m, b):
    asc = jnp.sum(xl * asv, axis=1, keepdims=True)
    adc = jnp.sum(xl * adv, axis=1, keepdims=True)
    es = asc + adc
    es = jnp.where(es > 0.0, es, 0.2 * es)
    exs = jnp.exp(es - m)
    return (acc + exs * xl) / (den + exs + 1e-16) + b


def _tc2_body(acc_ref, den_ref, xl_ref, as1_ref, ad1_ref, m1_ref, b1_ref,
              w_ref, asv_ref, adv_ref, xl2_ref, asr_ref, adr_ref, m2_ref):
    h = _norm_h(acc_ref[...], den_ref[...], xl_ref[...], as1_ref[...],
                ad1_ref[...], m1_ref[0, 0], b1_ref[...])
    h = jnp.where(h > 0.0, h, jnp.exp(jnp.minimum(h, 0.0)) - 1.0)
    xl2 = jnp.dot(h, w_ref[...], preferred_element_type=_f32)
    xl2_ref[...] = xl2
    _proj_tail(xl2, asv_ref[...], adv_ref[...], asr_ref, adr_ref, m2_ref,
               pl.program_id(0) == 0)


def _tc2(acc, den_b, xl1, a_src1, a_dst1, m1, b1, W2, a_src2, a_dst2):
    BM = 1000
    G = N // BM
    vec = pl.BlockSpec((1, H), lambda i: (0, 0))
    blk = pl.BlockSpec((BM, H), lambda i: (i, 0))
    return pl.pallas_call(
        _tc2_body,
        grid=(G,),
        in_specs=[blk, blk, blk, vec, vec,
                  pl.BlockSpec((1, 1), lambda i: (0, 0)), vec,
                  pl.BlockSpec((H, H), lambda i: (0, 0)), vec, vec],
        out_specs=[
            pl.BlockSpec((BM, H), lambda i: (i, 0)),
            pl.BlockSpec((1, 1, BM), lambda i: (i, 0, 0)),
            pl.BlockSpec((1, 1, BM), lambda i: (i, 0, 0)),
            pl.BlockSpec((1, 1), lambda i: (0, 0)),
        ],
        out_shape=[
            jax.ShapeDtypeStruct((N, H), _f32),
            jax.ShapeDtypeStruct((G, 1, BM), _f32),
            jax.ShapeDtypeStruct((G, 1, BM), _f32),
            jax.ShapeDtypeStruct((1, 1), _f32),
        ],
    )(acc, den_b, xl1, a_src1.reshape(1, H), a_dst1.reshape(1, H), m1,
      b1.reshape(1, H), W2, a_src2.reshape(1, H), a_dst2.reshape(1, H))


def _tc3_body(acc_ref, den_ref, xl_ref, as2_ref, ad2_ref, m2_ref, b2_ref,
              wp_ref, bp_ref, out_ref):
    h = _norm_h(acc_ref[...], den_ref[...], xl_ref[...], as2_ref[...],
                ad2_ref[...], m2_ref[0, 0], b2_ref[...])
    o = jnp.dot(h, wp_ref[...], preferred_element_type=_f32) + bp_ref[...]
    out_ref[...] = jnp.maximum(o, 0.0)


def _tc3(acc, den_b, xl2, a_src2, a_dst2, m2, b2, Wp, bp):
    BM = 1000
    G = N // BM
    vec = pl.BlockSpec((1, H), lambda i: (0, 0))
    blk = pl.BlockSpec((BM, H), lambda i: (i, 0))
    return pl.pallas_call(
        _tc3_body,
        grid=(G,),
        in_specs=[blk, blk, blk, vec, vec,
                  pl.BlockSpec((1, 1), lambda i: (0, 0)), vec,
                  pl.BlockSpec((H, H), lambda i: (0, 0)), vec],
        out_specs=pl.BlockSpec((BM, H), lambda i: (i, 0)),
        out_shape=jax.ShapeDtypeStruct((N, H), _f32),
    )(acc, den_b, xl2, a_src2.reshape(1, H), a_dst2.reshape(1, H), m2,
      b2.reshape(1, H), Wp, bp.reshape(1, H))


def _wid():
    return lax.axis_index("s") * 2 + lax.axis_index("c")


def _filter_body(src_ref, dst_ref, slist_ref, dlist_ref, counts_ref,
                 sbufA, dbufA, sbufB, dbufB, stg_s, stg_d, cbuf, semA, semB):
    wid = _wid()
    lo = wid * RPT
    iot = lax.iota(_i32, 16)
    zero16 = jnp.zeros((16,), _i32)
    sent = jnp.full((16,), RPT, _i32)
    NCHF = E // CH
    NPF = NCHF // 2

    def _issue(ci, sb, db, sem):
        off = pl.multiple_of(ci * CH, 8)
        pltpu.async_copy(src_ref.at[pl.ds(off, CH)], sb, sem)
        pltpu.async_copy(dst_ref.at[pl.ds(off, CH)], db, sem)

    def _drain(sb, db, sem):
        pltpu.make_async_copy(src_ref.at[pl.ds(0, CH)], sb, sem).wait()
        pltpu.make_async_copy(dst_ref.at[pl.ds(0, CH)], db, sem).wait()

    def _flush(cl, wt):
        fo = pl.multiple_of(wid * LCAP + wt, 8)
        pltpu.sync_copy(stg_s.at[pl.ds(0, FL)], slist_ref.at[pl.ds(fo, FL)])
        pltpu.sync_copy(stg_d.at[pl.ds(0, FL)], dlist_ref.at[pl.ds(fo, FL)])

        @plsc.parallel_loop(0, FL // 16, 1, unroll=4)
        def _mv(t):
            idx = jnp.full((16,), FL + t * 16, _i32) + iot
            dstx = jnp.full((16,), t * 16, _i32) + iot
            plsc.store_scatter(stg_s, [dstx], plsc.load_gather(stg_s, [idx]))
            plsc.store_scatter(stg_d, [dstx], plsc.load_gather(stg_d, [idx]))
        return cl - FL, wt + FL

    def _noflush(cl, wt):
        return cl, wt

    def _append(sb, db, cl, wt):
        clv = jnp.full((16,), cl, _i32)

        def _t(t, clv):
            t80 = jnp.full((16,), t * 80, _i32)
            for u in range(5):
                idx = t80 + (u * 16 + iot)
                sv = plsc.load_gather(sb, [idx])
                dv = plsc.load_gather(db, [idx])
                dloc = dv - lo
                msk = (dloc >= 0) & (dloc < RPT)
                pos = clv + plsc.cumsum(msk.astype(_i32)) - 1
                plsc.store_scatter(stg_s, [pos], sv, mask=msk)
                plsc.store_scatter(stg_d, [pos], dloc, mask=msk)
                clv = clv + plsc.all_reduce_population_count(msk)
            return clv

        clv = plsc.parallel_loop(0, CH // 80, 1, unroll=2, carry=clv)(_t)
        cl = jnp.max(clv)
        return lax.cond(cl >= FL, _flush, _noflush, cl, wt)

    _issue(0, sbufA, dbufA, semA)
    _issue(1, sbufB, dbufB, semB)

    def _pair(p, carry):
        cl, wt = carry
        _drain(sbufA, dbufA, semA)
        cl, wt = _append(sbufA, dbufA, cl, wt)

        @pl.when(2 * p + 2 < NCHF)
        def _():
            _issue(2 * p + 2, sbufA, dbufA, semA)

        _drain(sbufB, dbufB, semB)
        cl, wt = _append(sbufB, dbufB, cl, wt)

        @pl.when(2 * p + 3 < NCHF)
        def _():
            _issue(2 * p + 3, sbufB, dbufB, semB)

        return cl, wt

    cl, wt = lax.fori_loop(0, NPF, _pair, (jnp.int32(0), jnp.int32(0)))
    tcount = wt + cl
    for t in range(SENT // 16):
        pos = cl + t * 16 + iot
        plsc.store_scatter(stg_s, [pos], zero16)
        plsc.store_scatter(stg_d, [pos], sent)
    cl = cl + SENT
    cl, wt = lax.cond(cl >= FL, _flush, _noflush, cl, wt)
    fo = pl.multiple_of(wid * LCAP + wt, 8)
    pltpu.sync_copy(stg_s.at[pl.ds(0, FL)], slist_ref.at[pl.ds(fo, FL)])
    pltpu.sync_copy(stg_d.at[pl.ds(0, FL)], dlist_ref.at[pl.ds(fo, FL)])
    cbuf[...] = jnp.where(iot == 0, jnp.full((16,), tcount, _i32), 0)
    pltpu.sync_copy(cbuf, counts_ref.at[pl.ds(pl.multiple_of(wid * 16, 8), 16)])


@functools.partial(
    pl.kernel,
    out_type=[
        jax.ShapeDtypeStruct((NT * LCAP,), _i32),
        jax.ShapeDtypeStruct((NT * LCAP,), _i32),
        jax.ShapeDtypeStruct((NT * 16,), _i32),
    ],
    mesh=plsc.VectorSubcoreMesh(core_axis_name="c", subcore_axis_name="s"),
    compiler_params=pltpu.CompilerParams(needs_layout_passes=False),
    scratch_types=[
        pltpu.VMEM((CH,), _i32),
        pltpu.VMEM((CH,), _i32),
        pltpu.VMEM((CH,), _i32),
        pltpu.VMEM((CH,), _i32),
        pltpu.VMEM((SS,), _i32),
        pltpu.VMEM((SS,), _i32),
        pltpu.VMEM((16,), _i32),
        pltpu.SemaphoreType.DMA,
        pltpu.SemaphoreType.DMA,
    ],
)
def _filter(src_ref, dst_ref, slist_ref, dlist_ref, counts_ref, *scr):
    _filter_body(src_ref, dst_ref, slist_ref, dlist_ref, counts_ref, *scr)


def _edge_body(slist_ref, dlist_ref, counts_ref, asq_ref, adq_ref, xl_ref,
               mv_ref, acc_ref, dens_ref,
               as_t, ad_t, mbuf, cbuf,
               slb0, dlb0, exb0, rows0, slb1, dlb1, exb1, rows1,
               accv, denv, semL0, semL1, semR0, semR1):
    wid = _wid()
    lo = wid * RPT
    iot = lax.iota(_i32, 16)
    lane0 = iot == 0
    zero16f = jnp.zeros((16,), _f32)
    offs = [jj * 16 + iot for jj in range(H // 16)]

    pltpu.sync_copy(asq_ref, as_t)
    pltpu.sync_copy(adq_ref, ad_t)
    pltpu.sync_copy(mv_ref, mbuf)
    pltpu.sync_copy(counts_ref.at[pl.ds(pl.multiple_of(wid * 16, 8), 16)], cbuf)

    @plsc.parallel_loop(0, ACCR * H // 16, 1, unroll=8)
    def _zero(i):
        plsc.store_scatter(accv, [jnp.full((16,), i * 16, _i32) + iot], zero16f)
    for t in range(ACCR // 16):
        denv[pl.ds(t * 16, 16)] = zero16f

    cnt = jnp.max(plsc.load_gather(cbuf, [jnp.zeros((16,), _i32)]))
    nch = (cnt + (KE - 1)) // KE
    npair = (nch + 1) // 2
    mval = mbuf[...]

    def _issue_lists(ci, sb, db, sem):
        base = pl.multiple_of(wid * LCAP + ci * KE, 8)
        pltpu.async_copy(slist_ref.at[pl.ds(base, KE)], sb, sem)
        pltpu.async_copy(dlist_ref.at[pl.ds(base, KE)], db, sem)

    def _drain_lists(sb, db, sem):
        pltpu.make_async_copy(slist_ref.at[pl.ds(0, KE)], sb, sem).wait()
        pltpu.make_async_copy(dlist_ref.at[pl.ds(0, KE)], db, sem).wait()

    def _issue_rows(sb, rows, sem):
        pltpu.async_copy(xl_ref.at[sb.at[pl.ds(0, KH)]],
                         rows.at[pl.ds(0, KH)], sem)
        pltpu.async_copy(xl_ref.at[sb.at[pl.ds(KH, KH)]],
                         rows.at[pl.ds(KH, KH)], sem)

    def _drain_rows(sb, rows, sem):
        pltpu.make_async_copy(xl_ref.at[sb.at[pl.ds(0, KH)]],
                              rows.at[pl.ds(0, KH)], sem).wait()
        pltpu.make_async_copy(xl_ref.at[sb.at[pl.ds(KH, KH)]],
                              rows.at[pl.ds(KH, KH)], sem).wait()

    def _compute(sb, db, eb, rows):
        for t in range(KE // 16):
            sv = sb[pl.ds(t * 16, 16)]
            dv = db[pl.ds(t * 16, 16)]
            asg = plsc.load_gather(as_t, [sv])
            adg = plsc.load_gather(ad_t, [dv + lo])
            e = asg + adg
            e = jnp.where(e > 0.0, e, 0.2 * e)
            eb[pl.ds(t * 16, 16)] = jnp.exp(e - mval)

        @plsc.parallel_loop(0, KE, 1, unroll=4)
        def _edge1(j):
            js = jnp.full((16,), j, _i32)
            dls = plsc.load_gather(db, [js])
            exs = plsc.load_gather(eb, [js])
            plsc.addupdate_scatter(denv, [dls], exs, mask=lane0)
            rbase = dls * H
            for jj in range(H // 16):
                rv = plsc.load_gather(rows, [js, offs[jj]])
                plsc.addupdate_scatter(accv, [rbase + offs[jj]], rv * exs)

    _issue_lists(0, slb0, dlb0, semL0)
    _drain_lists(slb0, dlb0, semL0)
    _issue_rows(slb0, rows0, semR0)
    _issue_lists(1, slb1, dlb1, semL1)

    def _pair(p, _):
        cA = 2 * p
        _drain_rows(slb0, rows0, semR0)
        _drain_lists(slb1, dlb1, semL1)
        _issue_rows(slb1, rows1, semR1)
        _compute(slb0, dlb0, exb0, rows0)
        _issue_lists(cA + 2, slb0, dlb0, semL0)
        _drain_rows(slb1, rows1, semR1)
        _drain_lists(slb0, dlb0, semL0)
        _issue_rows(slb0, rows0, semR0)
        _compute(slb1, dlb1, exb1, rows1)
        _issue_lists(cA + 3, slb1, dlb1, semL1)
        return 0

    lax.fori_loop(0, npair, _pair, 0)
    _drain_rows(slb0, rows0, semR0)
    _drain_lists(slb1, dlb1, semL1)

    pltpu.sync_copy(accv.at[pl.ds(0, RPT * H)],
                    acc_ref.at[pl.ds(pl.multiple_of(lo * H, 8), RPT * H)])
    pltpu.sync_copy(denv, dens_ref.at[pl.ds(pl.multiple_of(wid * ACCR, 8), ACCR)])


@functools.partial(
    pl.kernel,
    out_type=[
        jax.ShapeDtypeStruct((NPAD * H,), _f32),
        jax.ShapeDtypeStruct((NT * ACCR,), _f32),
    ],
    mesh=plsc.VectorSubcoreMesh(core_axis_name="c", subcore_axis_name="s"),
    compiler_params=pltpu.CompilerParams(needs_layout_passes=False),
    scratch_types=[
        pltpu.VMEM((TPAD,), _f32),
        pltpu.VMEM((TPAD,), _f32),
        pltpu.VMEM((16,), _f32),
        pltpu.VMEM((16,), _i32),
        pltpu.VMEM((KE,), _i32),
        pltpu.VMEM((KE,), _i32),
        pltpu.VMEM((KE,), _f32),
        pltpu.VMEM((KE, H), _f32),
        pltpu.VMEM((KE,), _i32),
        pltpu.VMEM((KE,), _i32),
        pltpu.VMEM((KE,), _f32),
        pltpu.VMEM((KE, H), _f32),
        pltpu.VMEM((ACCR * H,), _f32),
        pltpu.VMEM((ACCR,), _f32),
        pltpu.SemaphoreType.DMA,
        pltpu.SemaphoreType.DMA,
        pltpu.SemaphoreType.DMA,
        pltpu.SemaphoreType.DMA,
    ],
)
def _edge_pass(slist_ref, dlist_ref, counts_ref, asq_ref, adq_ref, xl_ref,
               mv_ref, acc_ref, dens_ref, *scr):
    _edge_body(slist_ref, dlist_ref, counts_ref, asq_ref, adq_ref, xl_ref,
               mv_ref, acc_ref, dens_ref, *scr)


def _pad_table(v):
    return jnp.pad(v.reshape(-1), (0, TPAD - N))


def _sc_layer(slist, dlist, counts, asr, adr, xl, m):
    asq = _pad_table(asr)
    adq = _pad_table(adr)
    mv = jnp.broadcast_to(m.reshape(()), (16,))
    accf, densf = _edge_pass(slist, dlist, counts, asq, adq, xl, mv)
    acc = accf.reshape(NPAD, H)[:N]
    den = densf.reshape(NT, ACCR)[:, :RPT].reshape(NPAD)[:N]
    den_b = jnp.broadcast_to(den[:, None], (N, H))
    return acc, den_b


def kernel(x, edge_index, W1, a_src1, a_dst1, b1, W2, a_src2, a_dst2, b2,
           Wp, bp):
    src = edge_index[0]
    dst = edge_index[1]
    slist, dlist, counts = _filter(src, dst)

    xl1, asr1, adr1, m1 = _tc1(x, W1, a_src1, a_dst1)
    acc1, den1b = _sc_layer(slist, dlist, counts, asr1, adr1, xl1, m1)

    xl2, asr2, adr2, m2 = _tc2(acc1, den1b, xl1, a_src1, a_dst1, m1, b1,
                               W2, a_src2, a_dst2)
    acc2, den2b = _sc_layer(slist, dlist, counts, asr2, adr2, xl2, m2)

    return _tc3(acc2, den2b, xl2, a_src2, a_dst2, m2, b2, Wp, bp)
